# Initial kernel scaffold; baseline (speedup 1.0000x reference)
import jax
import jax.numpy as jnp
from jax.experimental import pallas as pl


def kernel(x, pos, pe, edge_index, batch, embed_W1, embed_b1, embed_W2, embed_b2, msg_W1, msg_b1, msg_W2, msg_b2, upd_W1, upd_b1, upd_W2, upd_b2, pre_W1, pre_b1, pre_W2, pre_b2, ro_W1, ro_b1, ro_W2, ro_b2):
    raise NotImplementedError("write your pallas kernel here")


# speedup vs baseline: 2.3955x; 2.3955x over previous
import functools

import jax
import jax.numpy as jnp
from jax import lax
from jax.experimental import pallas as pl
from jax.experimental.pallas import tpu as pltpu
from jax.experimental.pallas import tpu_sc as plsc

N = 10000
E = 320000
H = 128
G = 16

NC = 2
NS = 16
NW = NC * NS
EPW = E // NW
KC = 80
ICH = 2000
ROWS_PT = N // NS

NB = 1000
EB = 2000

_mesh = plsc.VectorSubcoreMesh(core_axis_name="c", subcore_axis_name="s")


def _silu(x):
    return x * jax.nn.sigmoid(x)


def _make_gather_sc(W, sign):
    @functools.partial(
        pl.kernel,
        mesh=_mesh,
        compiler_params=pltpu.CompilerParams(use_tc_tiling_on_sc=False),
        out_type=jax.ShapeDtypeStruct((E, W), jnp.float32),
        scratch_types=[
            pltpu.VMEM((ICH,), jnp.int32),
            pltpu.VMEM((ICH,), jnp.int32),
            pltpu.VMEM((KC, W), jnp.float32),
            pltpu.VMEM((KC, W), jnp.float32),
            pltpu.SemaphoreType.DMA,
            pltpu.SemaphoreType.DMA,
        ],
    )
    def gather_sc(a_hbm, b_hbm, send_hbm, rec_hbm, pre_hbm,
                  sidx, ridx, bufa, bufb, sema, semb):
        wid = lax.axis_index("s") * NC + lax.axis_index("c")
        base0 = wid * EPW
        groups = W // 16

        def chunk(c, _):
            base = pl.multiple_of(base0 + c * ICH, 8)
            pltpu.sync_copy(send_hbm.at[pl.ds(base, ICH)], sidx)
            pltpu.sync_copy(rec_hbm.at[pl.ds(base, ICH)], ridx)

            def inner(j, _):
                off = pl.multiple_of(j * KC, 8)
                da = pltpu.async_copy(a_hbm.at[sidx.at[pl.ds(off, KC)]], bufa, sema)
                db = pltpu.async_copy(b_hbm.at[ridx.at[pl.ds(off, KC)]], bufb, semb)
                da.wait()
                db.wait()

                def addrow(i, _):
                    r = i // groups
                    col = (i % groups) * 16
                    va = bufa[r, pl.ds(col, 16)]
                    vb = bufb[r, pl.ds(col, 16)]
                    bufa[r, pl.ds(col, 16)] = va + vb if sign > 0 else va - vb
                    return _

                lax.fori_loop(0, KC * groups, addrow, None)
                pltpu.sync_copy(bufa, pre_hbm.at[pl.ds(base + off, KC)])
                return _

            lax.fori_loop(0, ICH // KC, inner, None)
            return _

        lax.fori_loop(0, EPW // ICH, chunk, None)

    return gather_sc


_gather_sc = _make_gather_sc(H, 1)
_posdiff_sc = _make_gather_sc(16, -1)


@functools.partial(
    pl.kernel,
    mesh=_mesh,
    compiler_params=pltpu.CompilerParams(use_tc_tiling_on_sc=False),
    out_type=jax.ShapeDtypeStruct((NC, N, H), jnp.float32),
    scratch_types=[
        pltpu.VMEM_SHARED((N, H), jnp.float32),
        pltpu.VMEM((KC, H), jnp.float32),
        pltpu.VMEM((ICH // KC, KC), jnp.int32),
    ],
)
def _scatter_sc(m_hbm, rec2_hbm, zeros_hbm, agg_hbm, shared, mbuf, ridx2):
    core = lax.axis_index("c")
    sub = lax.axis_index("s")
    wid = sub * NC + core
    pltpu.sync_copy(zeros_hbm, shared.at[pl.ds(sub * ROWS_PT, ROWS_PT)])
    plsc.subcore_barrier()

    rows_per_worker = EPW // KC
    rows_per_chunk = ICH // KC

    def chunk(c, _):
        rowbase = wid * rows_per_worker + c * rows_per_chunk
        pltpu.sync_copy(rec2_hbm.at[pl.ds(rowbase, rows_per_chunk)], ridx2)

        def inner(j, _):
            ebase = pl.multiple_of((rowbase + j) * KC, 8)
            pltpu.sync_copy(m_hbm.at[pl.ds(ebase, KC)], mbuf)
            pltpu.sync_copy(mbuf, shared.at[ridx2.at[j]], add=True)
            return _

        lax.fori_loop(0, rows_per_chunk, inner, None)
        return _

    lax.fori_loop(0, rows_per_worker // rows_per_chunk, chunk, None)
    plsc.subcore_barrier()
    pltpu.sync_copy(
        shared.at[pl.ds(sub * ROWS_PT, ROWS_PT)],
        agg_hbm.at[core, pl.ds(sub * ROWS_PT, ROWS_PT)],
    )


def _d2_body(diff, od2):
    d = diff[...]
    od2[...] = jnp.sum(d * d, axis=1, keepdims=True)


def _embed_body(xc, w1, b1, w2, b2, o):
    t = _silu(jnp.dot(xc[...], w1[...], preferred_element_type=jnp.float32) + b1[...])
    o[...] = jnp.dot(t, w2[...], preferred_element_type=jnp.float32) + b2[...]


def _ab_body(h, wa, wb, oa, ob):
    hv = h[...]
    oa[...] = jnp.dot(hv, wa[...], preferred_element_type=jnp.float32)
    ob[...] = jnp.dot(hv, wb[...], preferred_element_type=jnp.float32)


def _edge_body(pre, d2, wd, b1, w2, b2, om):
    d2v = d2[...]
    dist = jnp.where(d2v > 0, jnp.sqrt(jnp.where(d2v > 0, d2v, 1.0)), 0.0)
    t = _silu(pre[...] + dist * wd[...] + b1[...])
    om[...] = _silu(jnp.dot(t, w2[...], preferred_element_type=jnp.float32) + b2[...])


def _node_body(h, p0, p1, u1a, u1b, ub1, u2, ub2, oh):
    hv = h[...]
    aggr = p0[0] + p1[0]
    t = _silu(
        jnp.dot(hv, u1a[...], preferred_element_type=jnp.float32)
        + jnp.dot(aggr, u1b[...], preferred_element_type=jnp.float32)
        + ub1[...]
    )
    oh[...] = hv + jnp.dot(t, u2[...], preferred_element_type=jnp.float32) + ub2[...]


def _final_body(h, bt, pw1, pb1, pw2, pb2, rw1, rb1, rw2, rb2, o, acc):
    i = pl.program_id(0)
    t = _silu(jnp.dot(h[...], pw1[...], preferred_element_type=jnp.float32) + pb1[...])
    z = jnp.dot(t, pw2[...], preferred_element_type=jnp.float32) + pb2[...]
    oh = (bt[...] == lax.broadcasted_iota(jnp.int32, (NB, G), 1)).astype(jnp.float32)
    part = lax.dot_general(oh, z, (((0,), (0,)), ((), ())),
                           preferred_element_type=jnp.float32)

    @pl.when(i == 0)
    def _():
        acc[...] = part

    @pl.when(i > 0)
    def _():
        acc[...] = acc[...] + part

    @pl.when(i == pl.num_programs(0) - 1)
    def _():
        tp = _silu(jnp.dot(acc[...], rw1[...], preferred_element_type=jnp.float32)
                   + rb1[...])
        o[...] = jnp.dot(tp, rw2[...], preferred_element_type=jnp.float32) + rb2[...]


def _full(r, c):
    return pl.BlockSpec((r, c), lambda i: (0, 0))


def kernel(x, pos, pe, edge_index, batch, embed_W1, embed_b1, embed_W2, embed_b2,
           msg_W1, msg_b1, msg_W2, msg_b2, upd_W1, upd_b1, upd_W2, upd_b2,
           pre_W1, pre_b1, pre_W2, pre_b2, ro_W1, ro_b1, ro_W2, ro_b2):
    L = msg_W1.shape[0]
    send = edge_index[0]
    rec = edge_index[1]
    rec2 = rec.reshape(E // KC, KC)
    posq = jnp.pad(pos, ((0, 0), (0, 13)))
    xcat = jnp.concatenate([x, pe], axis=-1)
    zeros = jnp.zeros((ROWS_PT, H), jnp.float32)

    diff = _posdiff_sc(posq, posq, send, rec)
    d2c = pl.pallas_call(
        _d2_body,
        grid=(E // EB,),
        in_specs=[pl.BlockSpec((EB, 16), lambda i: (i, 0))],
        out_specs=pl.BlockSpec((EB, 1), lambda i: (i, 0)),
        out_shape=jax.ShapeDtypeStruct((E, 1), jnp.float32),
    )(diff)

    h = pl.pallas_call(
        _embed_body,
        grid=(N // NB,),
        in_specs=[
            pl.BlockSpec((NB, H), lambda i: (i, 0)),
            _full(H, H), _full(1, H), _full(H, H), _full(1, H),
        ],
        out_specs=pl.BlockSpec((NB, H), lambda i: (i, 0)),
        out_shape=jax.ShapeDtypeStruct((N, H), jnp.float32),
    )(xcat, embed_W1, embed_b1.reshape(1, H), embed_W2, embed_b2.reshape(1, H))

    for l in range(L):
        a, b = pl.pallas_call(
            _ab_body,
            grid=(N // NB,),
            in_specs=[
                pl.BlockSpec((NB, H), lambda i: (i, 0)),
                _full(H, H), _full(H, H),
            ],
            out_specs=[
                pl.BlockSpec((NB, H), lambda i: (i, 0)),
                pl.BlockSpec((NB, H), lambda i: (i, 0)),
            ],
            out_shape=[
                jax.ShapeDtypeStruct((N, H), jnp.float32),
                jax.ShapeDtypeStruct((N, H), jnp.float32),
            ],
        )(h, msg_W1[l, :H], msg_W1[l, H:2 * H])

        pre = _gather_sc(a, b, send, rec)

        m = pl.pallas_call(
            _edge_body,
            grid=(E // EB,),
            in_specs=[
                pl.BlockSpec((EB, H), lambda i: (i, 0)),
                pl.BlockSpec((EB, 1), lambda i: (i, 0)),
                _full(1, H), _full(1, H), _full(H, H), _full(1, H),
            ],
            out_specs=pl.BlockSpec((EB, H), lambda i: (i, 0)),
            out_shape=jax.ShapeDtypeStruct((E, H), jnp.float32),
        )(pre, d2c, msg_W1[l, 2 * H:2 * H + 1], msg_b1[l].reshape(1, H),
          msg_W2[l], msg_b2[l].reshape(1, H))

        agg = _scatter_sc(m, rec2, zeros)

        h = pl.pallas_call(
            _node_body,
            grid=(N // NB,),
            in_specs=[
                pl.BlockSpec((NB, H), lambda i: (i, 0)),
                pl.BlockSpec((1, NB, H), lambda i: (0, i, 0)),
                pl.BlockSpec((1, NB, H), lambda i: (1, i, 0)),
                _full(H, H), _full(H, H), _full(1, H), _full(H, H), _full(1, H),
            ],
            out_specs=pl.BlockSpec((NB, H), lambda i: (i, 0)),
            out_shape=jax.ShapeDtypeStruct((N, H), jnp.float32),
        )(h, agg, agg, upd_W1[l, :H], upd_W1[l, H:], upd_b1[l].reshape(1, H),
          upd_W2[l], upd_b2[l].reshape(1, H))

    out = pl.pallas_call(
        _final_body,
        grid=(N // NB,),
        in_specs=[
            pl.BlockSpec((NB, H), lambda i: (i, 0)),
            pl.BlockSpec((NB, 1), lambda i: (i, 0)),
            _full(H, H), _full(1, H), _full(H, H), _full(1, H),
            _full(H, H), _full(1, H), _full(H, 1), _full(1, 1),
        ],
        out_specs=_full(G, 1),
        out_shape=jax.ShapeDtypeStruct((G, 1), jnp.float32),
        scratch_shapes=[pltpu.VMEM((G, H), jnp.float32)],
    )(h, batch.reshape(N, 1), pre_W1, pre_b1.reshape(1, H), pre_W2,
      pre_b2.reshape(1, H), ro_W1, ro_b1.reshape(1, H), ro_W2,
      ro_b2.reshape(1, 1))

    return out.reshape(G)


# speedup vs baseline: 4.3977x; 1.8358x over previous
import functools

import jax
import jax.numpy as jnp
from jax import lax
from jax.experimental import pallas as pl
from jax.experimental.pallas import tpu as pltpu
from jax.experimental.pallas import tpu_sc as plsc

N = 10000
E = 320000
H = 128
G = 16

NC = 2
NS = 16
NW = NC * NS
EPW = E // NW
KC = 80
ICH = 2000
ROWS_PT = N // NS

NB = 1000
EB = 2000

_mesh = plsc.VectorSubcoreMesh(core_axis_name="c", subcore_axis_name="s")


def _silu(x):
    return x * jax.nn.sigmoid(x)


NBUF = 4
CPW = EPW // KC
MAINK = (CPW - 1) // NBUF


def _make_gather_sc(W, sign):
    groups = W // 16

    @functools.partial(
        pl.kernel,
        mesh=_mesh,
        compiler_params=pltpu.CompilerParams(use_tc_tiling_on_sc=False),
        out_type=jax.ShapeDtypeStruct((E, W), jnp.float32),
        scratch_types=[
            pltpu.VMEM((EPW,), jnp.int32),
            pltpu.VMEM((EPW,), jnp.int32),
        ]
        + [pltpu.VMEM((KC, W), jnp.float32) for _ in range(2 * NBUF + 2)]
        + [pltpu.SemaphoreType.DMA for _ in range(2 * NBUF + 2)],
    )
    def gather_sc(a_hbm, b_hbm, send_hbm, rec_hbm, pre_hbm, sidx, ridx, *scr):
        bufa = scr[0:NBUF]
        bufb = scr[NBUF:2 * NBUF]
        obuf = scr[2 * NBUF:2 * NBUF + 2]
        sema = scr[2 * NBUF + 2:3 * NBUF + 2]
        semb = scr[3 * NBUF + 2:4 * NBUF + 2]
        semw = scr[4 * NBUF + 2:4 * NBUF + 4]
        wid = lax.axis_index("s") * NC + lax.axis_index("c")
        base0 = wid * EPW
        pltpu.sync_copy(send_hbm.at[pl.ds(base0, EPW)], sidx)
        pltpu.sync_copy(rec_hbm.at[pl.ds(base0, EPW)], ridx)

        def fire(c, b):
            off = pl.multiple_of(c * KC, 8)
            pltpu.async_copy(a_hbm.at[sidx.at[pl.ds(off, KC)]], bufa[b], sema[b])
            pltpu.async_copy(b_hbm.at[ridx.at[pl.ds(off, KC)]], bufb[b], semb[b])

        def drain(c, b):
            off = pl.multiple_of(c * KC, 8)
            pltpu.make_async_copy(
                a_hbm.at[sidx.at[pl.ds(off, KC)]], bufa[b], sema[b]).wait()
            pltpu.make_async_copy(
                b_hbm.at[ridx.at[pl.ds(off, KC)]], bufb[b], semb[b]).wait()

        def combine(b, p):
            def addrow(r, _):
                for g in range(groups):
                    va = bufa[b][r, pl.ds(g * 16, 16)]
                    vb = bufb[b][r, pl.ds(g * 16, 16)]
                    obuf[p][r, pl.ds(g * 16, 16)] = va + vb if sign > 0 else va - vb
                return _

            lax.fori_loop(0, KC, addrow, None)

        def wb_fire(c, p):
            pltpu.async_copy(
                obuf[p], pre_hbm.at[pl.ds(base0 + c * KC, KC)], semw[p])

        def wb_drain(c, p):
            pltpu.make_async_copy(
                obuf[p], pre_hbm.at[pl.ds(base0 + c * KC, KC)], semw[p]).wait()

        for b in range(NBUF):
            fire(b, b)

        def ring(k, _):
            for b in range(NBUF):
                c = k * NBUF + b
                p = b % 2
                drain(c, b)
                if b >= 2:
                    wb_drain(c - 2, p)
                else:
                    @pl.when(k > 0)
                    def _():
                        wb_drain(c - 2, p)
                combine(b, p)
                wb_fire(c, p)

                @pl.when(k < MAINK - 1)
                def _():
                    fire(c + NBUF, b)
            return _

        lax.fori_loop(0, MAINK, ring, None)

        cl = CPW - 1
        fire(cl, 0)
        drain(cl, 0)
        wb_drain(cl - 2, 0)
        combine(0, 0)
        wb_fire(cl, 0)
        wb_drain(cl - 1, 1)
        wb_drain(cl, 0)

    return gather_sc


_gather_sc = _make_gather_sc(H, 1)
_posdiff_sc = _make_gather_sc(16, -1)


@functools.partial(
    pl.kernel,
    mesh=_mesh,
    compiler_params=pltpu.CompilerParams(use_tc_tiling_on_sc=False),
    out_type=jax.ShapeDtypeStruct((NC, N, H), jnp.float32),
    scratch_types=[
        pltpu.VMEM_SHARED((N, H), jnp.float32),
        pltpu.VMEM((CPW, KC), jnp.int32),
    ]
    + [pltpu.VMEM((KC, H), jnp.float32) for _ in range(NBUF)]
    + [pltpu.SemaphoreType.DMA for _ in range(NBUF)],
)
def _scatter_sc(m_hbm, rec2_hbm, zeros_hbm, agg_hbm, shared, ridx2, *scr):
    mbuf = scr[0:NBUF]
    sems = scr[NBUF:2 * NBUF]
    core = lax.axis_index("c")
    sub = lax.axis_index("s")
    wid = sub * NC + core
    rowbase = wid * CPW

    def fire(c, b):
        pltpu.async_copy(m_hbm.at[pl.ds((rowbase + c) * KC, KC)], mbuf[b], sems[b])

    def drain(c, b):
        pltpu.make_async_copy(
            m_hbm.at[pl.ds((rowbase + c) * KC, KC)], mbuf[b], sems[b]).wait()

    for b in range(NBUF):
        fire(b, b)
    pltpu.sync_copy(rec2_hbm.at[pl.ds(rowbase, CPW)], ridx2)
    pltpu.sync_copy(zeros_hbm, shared.at[pl.ds(sub * ROWS_PT, ROWS_PT)])
    plsc.subcore_barrier()

    def ring(k, _):
        for b in range(NBUF):
            c = k * NBUF + b
            drain(c, b)
            pltpu.sync_copy(mbuf[b], shared.at[ridx2.at[c]], add=True)

            @pl.when(k < MAINK - 1)
            def _():
                fire(c + NBUF, b)
        return _

    lax.fori_loop(0, MAINK, ring, None)
    cl = CPW - 1
    fire(cl, 0)
    drain(cl, 0)
    pltpu.sync_copy(mbuf[0], shared.at[ridx2.at[cl]], add=True)
    plsc.subcore_barrier()
    pltpu.sync_copy(
        shared.at[pl.ds(sub * ROWS_PT, ROWS_PT)],
        agg_hbm.at[core, pl.ds(sub * ROWS_PT, ROWS_PT)],
    )


def _d2_body(diff, od2):
    d = diff[...]
    od2[...] = jnp.sum(d * d, axis=1, keepdims=True)


def _embed_body(xc, w1, b1, w2, b2, o):
    t = _silu(jnp.dot(xc[...], w1[...], preferred_element_type=jnp.float32) + b1[...])
    o[...] = jnp.dot(t, w2[...], preferred_element_type=jnp.float32) + b2[...]


def _ab_body(h, wa, wb, oa, ob):
    hv = h[...]
    oa[...] = jnp.dot(hv, wa[...], preferred_element_type=jnp.float32)
    ob[...] = jnp.dot(hv, wb[...], preferred_element_type=jnp.float32)


def _edge_body(pre, d2, wd, b1, w2, b2, om):
    d2v = d2[...]
    dist = jnp.where(d2v > 0, jnp.sqrt(jnp.where(d2v > 0, d2v, 1.0)), 0.0)
    t = _silu(pre[...] + dist * wd[...] + b1[...])
    om[...] = _silu(jnp.dot(t, w2[...], preferred_element_type=jnp.float32) + b2[...])


def _node_body(h, p0, p1, u1a, u1b, ub1, u2, ub2, oh):
    hv = h[...]
    aggr = p0[0] + p1[0]
    t = _silu(
        jnp.dot(hv, u1a[...], preferred_element_type=jnp.float32)
        + jnp.dot(aggr, u1b[...], preferred_element_type=jnp.float32)
        + ub1[...]
    )
    oh[...] = hv + jnp.dot(t, u2[...], preferred_element_type=jnp.float32) + ub2[...]


def _final_body(h, bt, pw1, pb1, pw2, pb2, rw1, rb1, rw2, rb2, o, acc):
    i = pl.program_id(0)
    t = _silu(jnp.dot(h[...], pw1[...], preferred_element_type=jnp.float32) + pb1[...])
    z = jnp.dot(t, pw2[...], preferred_element_type=jnp.float32) + pb2[...]
    oh = (bt[...] == lax.broadcasted_iota(jnp.int32, (NB, G), 1)).astype(jnp.float32)
    part = lax.dot_general(oh, z, (((0,), (0,)), ((), ())),
                           preferred_element_type=jnp.float32)

    @pl.when(i == 0)
    def _():
        acc[...] = part

    @pl.when(i > 0)
    def _():
        acc[...] = acc[...] + part

    @pl.when(i == pl.num_programs(0) - 1)
    def _():
        tp = _silu(jnp.dot(acc[...], rw1[...], preferred_element_type=jnp.float32)
                   + rb1[...])
        o[...] = jnp.dot(tp, rw2[...], preferred_element_type=jnp.float32) + rb2[...]


def _full(r, c):
    return pl.BlockSpec((r, c), lambda i: (0, 0))


def kernel(x, pos, pe, edge_index, batch, embed_W1, embed_b1, embed_W2, embed_b2,
           msg_W1, msg_b1, msg_W2, msg_b2, upd_W1, upd_b1, upd_W2, upd_b2,
           pre_W1, pre_b1, pre_W2, pre_b2, ro_W1, ro_b1, ro_W2, ro_b2):
    L = msg_W1.shape[0]
    send = edge_index[0]
    rec = edge_index[1]
    rec2 = rec.reshape(E // KC, KC)
    posq = jnp.pad(pos, ((0, 0), (0, 13)))
    xcat = jnp.concatenate([x, pe], axis=-1)
    zeros = jnp.zeros((ROWS_PT, H), jnp.float32)

    diff = _posdiff_sc(posq, posq, send, rec)
    d2c = pl.pallas_call(
        _d2_body,
        grid=(E // EB,),
        in_specs=[pl.BlockSpec((EB, 16), lambda i: (i, 0))],
        out_specs=pl.BlockSpec((EB, 1), lambda i: (i, 0)),
        out_shape=jax.ShapeDtypeStruct((E, 1), jnp.float32),
    )(diff)

    h = pl.pallas_call(
        _embed_body,
        grid=(N // NB,),
        in_specs=[
            pl.BlockSpec((NB, H), lambda i: (i, 0)),
            _full(H, H), _full(1, H), _full(H, H), _full(1, H),
        ],
        out_specs=pl.BlockSpec((NB, H), lambda i: (i, 0)),
        out_shape=jax.ShapeDtypeStruct((N, H), jnp.float32),
    )(xcat, embed_W1, embed_b1.reshape(1, H), embed_W2, embed_b2.reshape(1, H))

    for l in range(L):
        a, b = pl.pallas_call(
            _ab_body,
            grid=(N // NB,),
            in_specs=[
                pl.BlockSpec((NB, H), lambda i: (i, 0)),
                _full(H, H), _full(H, H),
            ],
            out_specs=[
                pl.BlockSpec((NB, H), lambda i: (i, 0)),
                pl.BlockSpec((NB, H), lambda i: (i, 0)),
            ],
            out_shape=[
                jax.ShapeDtypeStruct((N, H), jnp.float32),
                jax.ShapeDtypeStruct((N, H), jnp.float32),
            ],
        )(h, msg_W1[l, :H], msg_W1[l, H:2 * H])

        pre = _gather_sc(a, b, send, rec)

        m = pl.pallas_call(
            _edge_body,
            grid=(E // EB,),
            in_specs=[
                pl.BlockSpec((EB, H), lambda i: (i, 0)),
                pl.BlockSpec((EB, 1), lambda i: (i, 0)),
                _full(1, H), _full(1, H), _full(H, H), _full(1, H),
            ],
            out_specs=pl.BlockSpec((EB, H), lambda i: (i, 0)),
            out_shape=jax.ShapeDtypeStruct((E, H), jnp.float32),
        )(pre, d2c, msg_W1[l, 2 * H:2 * H + 1], msg_b1[l].reshape(1, H),
          msg_W2[l], msg_b2[l].reshape(1, H))

        agg = _scatter_sc(m, rec2, zeros)

        h = pl.pallas_call(
            _node_body,
            grid=(N // NB,),
            in_specs=[
                pl.BlockSpec((NB, H), lambda i: (i, 0)),
                pl.BlockSpec((1, NB, H), lambda i: (0, i, 0)),
                pl.BlockSpec((1, NB, H), lambda i: (1, i, 0)),
                _full(H, H), _full(H, H), _full(1, H), _full(H, H), _full(1, H),
            ],
            out_specs=pl.BlockSpec((NB, H), lambda i: (i, 0)),
            out_shape=jax.ShapeDtypeStruct((N, H), jnp.float32),
        )(h, agg, agg, upd_W1[l, :H], upd_W1[l, H:], upd_b1[l].reshape(1, H),
          upd_W2[l], upd_b2[l].reshape(1, H))

    out = pl.pallas_call(
        _final_body,
        grid=(N // NB,),
        in_specs=[
            pl.BlockSpec((NB, H), lambda i: (i, 0)),
            pl.BlockSpec((NB, 1), lambda i: (i, 0)),
            _full(H, H), _full(1, H), _full(H, H), _full(1, H),
            _full(H, H), _full(1, H), _full(H, 1), _full(1, 1),
        ],
        out_specs=_full(G, 1),
        out_shape=jax.ShapeDtypeStruct((G, 1), jnp.float32),
        scratch_shapes=[pltpu.VMEM((G, H), jnp.float32)],
    )(h, batch.reshape(N, 1), pre_W1, pre_b1.reshape(1, H), pre_W2,
      pre_b2.reshape(1, H), ro_W1, ro_b1.reshape(1, H), ro_W2,
      ro_b2.reshape(1, 1))

    return out.reshape(G)
